# Initial kernel scaffold; baseline (speedup 1.0000x reference)
import jax
import jax.numpy as jnp
from jax.experimental import pallas as pl


def kernel(x, edge_index, batch_index, mol_features, gcn0_W, gcn0_b, gcn1_W, gcn1_b, gcn2_W, gcn2_b, gcnlin_W, gcnlin_b, mlp0_W, mlp0_b, mlp1_W, mlp1_b, mlp2_W, mlp2_b, pred0_W, pred0_b, pred1_W, pred1_b, out_W, out_b):
    raise NotImplementedError("write your pallas kernel here")


# speedup vs baseline: 8.1162x; 8.1162x over previous
import functools

import jax
import jax.numpy as jnp
from jax import lax
from jax.experimental import pallas as pl
from jax.experimental.pallas import tpu as pltpu
from jax.experimental.pallas import tpu_sc as plsc

N = 10000
E = 320000
B = 64
D_IN = 128
D_HID = 256
D_GOUT = 128
M_IN = 200
M_HID = 256
M_OUT = 64
P_HID = 256

NC = 2
NS = 16
NW = NC * NS
LANES = 16

BLK = 1000
GRID_N = N // BLK
HALF = D_HID // 2
EP_TILE = E // NS
K_EDGE = 80
N_CHUNK = EP_TILE // K_EDGE
EP_DEG = E // NW
NPAD = 10240
ROWS_PER_TILE = NPAD // NS

_mesh = plsc.VectorSubcoreMesh(
    core_axis_name="c", subcore_axis_name="s", num_cores=NC, num_subcores=NS
)


@functools.partial(
    pl.kernel,
    out_type=jax.ShapeDtypeStruct((NW * N,), jnp.float32),
    mesh=_mesh,
    scratch_types=[
        pltpu.VMEM((EP_DEG,), jnp.int32),
        pltpu.VMEM((N,), jnp.float32),
    ],
    compiler_params=pltpu.CompilerParams(needs_layout_passes=False),
)
def _sc_degree(dst_hbm, zvec_hbm, out_hbm, idx_v, cnt_v):
    wid = lax.axis_index("s") * NC + lax.axis_index("c")
    pltpu.sync_copy(zvec_hbm, cnt_v)
    pltpu.sync_copy(dst_hbm.at[pl.ds(wid * EP_DEG, EP_DEG)], idx_v)
    ones = jnp.ones((LANES,), jnp.float32)

    def body(i, _):
        idx = idx_v[pl.ds(i * LANES, LANES)]
        plsc.addupdate_scatter(cnt_v, [idx], ones)
        return 0

    lax.fori_loop(0, EP_DEG // LANES, body, 0)
    pltpu.sync_copy(cnt_v, out_hbm.at[pl.ds(wid * N, N)])


@functools.partial(
    pl.kernel,
    out_type=(
        jax.ShapeDtypeStruct((NPAD, HALF), jnp.float32),
        jax.ShapeDtypeStruct((NPAD, HALF), jnp.float32),
    ),
    mesh=_mesh,
    scratch_types=[
        pltpu.VMEM_SHARED((NPAD, HALF), jnp.float32),
        pltpu.VMEM((K_EDGE,), jnp.int32),
        pltpu.VMEM((K_EDGE,), jnp.int32),
        pltpu.VMEM((K_EDGE, HALF), jnp.float32),
        pltpu.SemaphoreType.DMA,
    ],
    compiler_params=pltpu.CompilerParams(needs_layout_passes=False),
)
def _sc_scatter(g_lo, g_hi, src_hbm, dst_hbm, zrows_hbm,
                s_lo, s_hi, acc, idxs, idxd, rows, sem):
    c = lax.axis_index("c")
    t = lax.axis_index("s")
    pltpu.sync_copy(zrows_hbm, acc.at[pl.ds(t * ROWS_PER_TILE, ROWS_PER_TILE)])
    plsc.subcore_barrier()

    ebase = t * EP_TILE

    def run(g_ref):
        def body(i, _):
            off = ebase + i * K_EDGE
            pltpu.sync_copy(src_hbm.at[pl.ds(off, K_EDGE)], idxs)
            pltpu.sync_copy(dst_hbm.at[pl.ds(off, K_EDGE)], idxd)
            pltpu.async_copy(g_ref.at[idxs], rows, sem).wait()
            pltpu.sync_copy(rows, acc.at[idxd], add=True)
            return 0

        lax.fori_loop(0, N_CHUNK, body, 0)

    @pl.when(c == 0)
    def _():
        run(g_lo)

    @pl.when(c == 1)
    def _():
        run(g_hi)

    plsc.subcore_barrier()
    rb = pl.ds(t * ROWS_PER_TILE, ROWS_PER_TILE)

    @pl.when(c == 0)
    def _():
        pltpu.sync_copy(acc.at[rb], s_lo.at[rb])

    @pl.when(c == 1)
    def _():
        pltpu.sync_copy(acc.at[rb], s_hi.at[rb])


def _tc_pre_body(x_ref, cnt_ref, w_ref, glo_ref, ghi_ref, dinv_ref):
    deg = jnp.sum(cnt_ref[...], axis=1, keepdims=True) + 1.0
    dinv = lax.rsqrt(deg)
    h = jnp.dot(x_ref[...], w_ref[...], preferred_element_type=jnp.float32)
    g = dinv * h
    glo_ref[...] = g[:, :HALF]
    ghi_ref[...] = g[:, HALF:]
    dinv_ref[...] = dinv


def _tc_pre(x, counts_t, w0):
    return pl.pallas_call(
        _tc_pre_body,
        grid=(GRID_N,),
        in_specs=[
            pl.BlockSpec((BLK, D_IN), lambda i: (i, 0)),
            pl.BlockSpec((BLK, NW), lambda i: (i, 0)),
            pl.BlockSpec((D_IN, D_HID), lambda i: (0, 0)),
        ],
        out_specs=[
            pl.BlockSpec((BLK, HALF), lambda i: (i, 0)),
            pl.BlockSpec((BLK, HALF), lambda i: (i, 0)),
            pl.BlockSpec((BLK, 1), lambda i: (i, 0)),
        ],
        out_shape=[
            jax.ShapeDtypeStruct((N, HALF), jnp.float32),
            jax.ShapeDtypeStruct((N, HALF), jnp.float32),
            jax.ShapeDtypeStruct((N, 1), jnp.float32),
        ],
    )(x, counts_t, w0)


def _tc_mid_body(slo_ref, shi_ref, glo_ref, ghi_ref, dinv_ref, b_ref, w_ref,
                 olo_ref, ohi_ref):
    dinv = dinv_ref[...]
    lo = dinv * (slo_ref[...] + glo_ref[...]) + b_ref[0, :HALF]
    hi = dinv * (shi_ref[...] + ghi_ref[...]) + b_ref[0, HALF:]
    t = jnp.concatenate([lo, hi], axis=1)
    h = jnp.dot(t, w_ref[...], preferred_element_type=jnp.float32)
    g = dinv * h
    olo_ref[...] = g[:, :HALF]
    ohi_ref[...] = g[:, HALF:]


def _tc_mid(s_lo, s_hi, g_lo, g_hi, dinv, b_prev, w_next):
    return pl.pallas_call(
        _tc_mid_body,
        grid=(GRID_N,),
        in_specs=[
            pl.BlockSpec((BLK, HALF), lambda i: (i, 0)),
            pl.BlockSpec((BLK, HALF), lambda i: (i, 0)),
            pl.BlockSpec((BLK, HALF), lambda i: (i, 0)),
            pl.BlockSpec((BLK, HALF), lambda i: (i, 0)),
            pl.BlockSpec((BLK, 1), lambda i: (i, 0)),
            pl.BlockSpec((1, D_HID), lambda i: (0, 0)),
            pl.BlockSpec((D_HID, D_HID), lambda i: (0, 0)),
        ],
        out_specs=[
            pl.BlockSpec((BLK, HALF), lambda i: (i, 0)),
            pl.BlockSpec((BLK, HALF), lambda i: (i, 0)),
        ],
        out_shape=[
            jax.ShapeDtypeStruct((N, HALF), jnp.float32),
            jax.ShapeDtypeStruct((N, HALF), jnp.float32),
        ],
    )(s_lo, s_hi, g_lo, g_hi, dinv, b_prev, w_next)


def _tc_post_body(slo_ref, shi_ref, glo_ref, ghi_ref, dinv_ref, b_ref,
                  wlin_ref, blin_ref, bi_ref, h1_ref, psum, pcnt):
    i = pl.program_id(0)

    @pl.when(i == 0)
    def _():
        psum[...] = jnp.zeros_like(psum)
        pcnt[...] = jnp.zeros_like(pcnt)

    dinv = dinv_ref[...]
    lo = dinv * (slo_ref[...] + glo_ref[...]) + b_ref[0, :HALF]
    hi = dinv * (shi_ref[...] + ghi_ref[...]) + b_ref[0, HALF:]
    t = jnp.concatenate([lo, hi], axis=1)
    z = jnp.dot(t, wlin_ref[...], preferred_element_type=jnp.float32)
    z = jnp.maximum(z + blin_ref[0, :], 0.0)
    iota = lax.broadcasted_iota(jnp.int32, (BLK, B), 1)
    oh = (bi_ref[...] == iota).astype(jnp.float32)
    psum[...] += lax.dot_general(
        oh, z, (((0,), (0,)), ((), ())), preferred_element_type=jnp.float32
    )
    pcnt[...] += lax.dot_general(
        oh, jnp.ones((BLK, 1), jnp.float32), (((0,), (0,)), ((), ())),
        preferred_element_type=jnp.float32,
    )

    @pl.when(i == GRID_N - 1)
    def _():
        h1_ref[...] = psum[...] / jnp.maximum(pcnt[...], 1.0)


def _tc_post(s_lo, s_hi, g_lo, g_hi, dinv, b2, wlin, blin, bi):
    return pl.pallas_call(
        _tc_post_body,
        grid=(GRID_N,),
        in_specs=[
            pl.BlockSpec((BLK, HALF), lambda i: (i, 0)),
            pl.BlockSpec((BLK, HALF), lambda i: (i, 0)),
            pl.BlockSpec((BLK, HALF), lambda i: (i, 0)),
            pl.BlockSpec((BLK, HALF), lambda i: (i, 0)),
            pl.BlockSpec((BLK, 1), lambda i: (i, 0)),
            pl.BlockSpec((1, D_HID), lambda i: (0, 0)),
            pl.BlockSpec((D_HID, D_GOUT), lambda i: (0, 0)),
            pl.BlockSpec((1, D_GOUT), lambda i: (0, 0)),
            pl.BlockSpec((BLK, 1), lambda i: (i, 0)),
        ],
        out_specs=pl.BlockSpec((B, D_GOUT), lambda i: (0, 0)),
        out_shape=jax.ShapeDtypeStruct((B, D_GOUT), jnp.float32),
        scratch_shapes=[
            pltpu.VMEM((B, D_GOUT), jnp.float32),
            pltpu.VMEM((B, 1), jnp.float32),
        ],
    )(s_lo, s_hi, g_lo, g_hi, dinv, b2, wlin, blin, bi)


def _tc_head_body(h1_ref, mf_ref, m0w, m0b, m1w, m1b, m2w, m2b,
                  p0wg, p0wm, p0b, p1w, p1b, ow, ob, out_ref):
    h2 = jnp.maximum(
        jnp.dot(mf_ref[...], m0w[...], preferred_element_type=jnp.float32)
        + m0b[0, :], 0.0)
    h2 = jnp.maximum(
        jnp.dot(h2, m1w[...], preferred_element_type=jnp.float32)
        + m1b[0, :], 0.0)
    h2 = jnp.maximum(
        jnp.dot(h2, m2w[...], preferred_element_type=jnp.float32)
        + m2b[0, :], 0.0)
    h = (jnp.dot(h1_ref[...], p0wg[...], preferred_element_type=jnp.float32)
         + jnp.dot(h2, p0wm[...], preferred_element_type=jnp.float32)
         + p0b[0, :])
    h = jnp.maximum(h, 0.0)
    h = jnp.maximum(
        jnp.dot(h, p1w[...], preferred_element_type=jnp.float32)
        + p1b[0, :], 0.0)
    out_ref[...] = (
        jnp.dot(h, ow[...], preferred_element_type=jnp.float32) + ob[0, :])


def _tc_head(h1, mf, m0w, m0b, m1w, m1b, m2w, m2b, p0wg, p0wm, p0b,
             p1w, p1b, ow, ob):
    args = (h1, mf, m0w, m0b, m1w, m1b, m2w, m2b, p0wg, p0wm, p0b,
            p1w, p1b, ow, ob)
    return pl.pallas_call(
        _tc_head_body,
        in_specs=[pl.BlockSpec(a.shape, lambda: (0, 0)) for a in args],
        out_specs=pl.BlockSpec((B, 1), lambda: (0, 0)),
        out_shape=jax.ShapeDtypeStruct((B, 1), jnp.float32),
    )(*args)


def kernel(x, edge_index, batch_index, mol_features,
           gcn0_W, gcn0_b, gcn1_W, gcn1_b, gcn2_W, gcn2_b,
           gcnlin_W, gcnlin_b,
           mlp0_W, mlp0_b, mlp1_W, mlp1_b, mlp2_W, mlp2_b,
           pred0_W, pred0_b, pred1_W, pred1_b, out_W, out_b):
    src = edge_index[0]
    dst = edge_index[1]
    bi = batch_index.reshape(N, 1).astype(jnp.int32)
    zvec = jnp.zeros((N,), jnp.float32)
    zrows = jnp.zeros((ROWS_PER_TILE, HALF), jnp.float32)

    counts = _sc_degree(dst, zvec)
    g_lo, g_hi, dinv = _tc_pre(x, counts.reshape(NW, N).T, gcn0_W)

    for w_next, b_prev in ((gcn1_W, gcn0_b), (gcn2_W, gcn1_b)):
        s_lo, s_hi = _sc_scatter(g_lo, g_hi, src, dst, zrows)
        g_lo, g_hi = _tc_mid(s_lo, s_hi, g_lo, g_hi, dinv,
                             b_prev.reshape(1, -1), w_next)

    s_lo, s_hi = _sc_scatter(g_lo, g_hi, src, dst, zrows)
    h1 = _tc_post(s_lo, s_hi, g_lo, g_hi, dinv, gcn2_b.reshape(1, -1),
                  gcnlin_W, gcnlin_b.reshape(1, -1), bi)

    return _tc_head(
        h1, mol_features,
        mlp0_W, mlp0_b.reshape(1, -1), mlp1_W, mlp1_b.reshape(1, -1),
        mlp2_W, mlp2_b.reshape(1, -1),
        pred0_W[:D_GOUT], pred0_W[D_GOUT:], pred0_b.reshape(1, -1),
        pred1_W, pred1_b.reshape(1, -1), out_W, out_b.reshape(1, -1))


# speedup vs baseline: 19.6206x; 2.4174x over previous
import functools

import jax
import jax.numpy as jnp
from jax import lax
from jax.experimental import pallas as pl
from jax.experimental.pallas import tpu as pltpu
from jax.experimental.pallas import tpu_sc as plsc

N = 10000
E = 320000
B = 64
D_IN = 128
D_HID = 256
D_GOUT = 128
M_IN = 200
M_HID = 256
M_OUT = 64
P_HID = 256

NC = 2
NS = 16
NW = NC * NS
LANES = 16

BLK = 1000
GRID_N = N // BLK
HALF = D_HID // 2
EP_TILE = E // NS
K_EDGE = 80
SC_E = 4000
N_SUPER = EP_TILE // SC_E
CPS = SC_E // K_EDGE
EP_DEG = E // NW
NPAD = 10240
ROWS_PER_TILE = NPAD // NS

_mesh = plsc.VectorSubcoreMesh(
    core_axis_name="c", subcore_axis_name="s", num_cores=NC, num_subcores=NS
)


@functools.partial(
    pl.kernel,
    out_type=jax.ShapeDtypeStruct((NW * N,), jnp.float32),
    mesh=_mesh,
    scratch_types=[
        pltpu.VMEM((EP_DEG,), jnp.int32),
        pltpu.VMEM((N,), jnp.float32),
    ],
    compiler_params=pltpu.CompilerParams(needs_layout_passes=False),
)
def _sc_degree(dst_hbm, zvec_hbm, out_hbm, idx_v, cnt_v):
    wid = lax.axis_index("s") * NC + lax.axis_index("c")
    pltpu.sync_copy(zvec_hbm, cnt_v)
    pltpu.sync_copy(dst_hbm.at[pl.ds(wid * EP_DEG, EP_DEG)], idx_v)
    ones = jnp.ones((LANES,), jnp.float32)

    def body(i, _):
        idx = idx_v[pl.ds(i * LANES, LANES)]
        plsc.addupdate_scatter(cnt_v, [idx], ones)
        return 0

    lax.fori_loop(0, EP_DEG // LANES, body, 0)
    pltpu.sync_copy(cnt_v, out_hbm.at[pl.ds(wid * N, N)])


@functools.partial(
    pl.kernel,
    out_type=(
        jax.ShapeDtypeStruct((NPAD, HALF), jnp.float32),
        jax.ShapeDtypeStruct((NPAD, HALF), jnp.float32),
    ),
    mesh=_mesh,
    scratch_types=[
        pltpu.VMEM_SHARED((NPAD, HALF), jnp.float32),
        pltpu.VMEM((SC_E,), jnp.int32),
        pltpu.VMEM((SC_E,), jnp.int32),
        pltpu.VMEM((SC_E,), jnp.int32),
        pltpu.VMEM((SC_E,), jnp.int32),
        pltpu.VMEM((K_EDGE,), jnp.int32),
        pltpu.VMEM((K_EDGE,), jnp.int32),
        pltpu.VMEM((K_EDGE, HALF), jnp.float32),
        pltpu.VMEM((K_EDGE, HALF), jnp.float32),
        pltpu.SemaphoreType.DMA,
        pltpu.SemaphoreType.DMA,
        pltpu.SemaphoreType.DMA,
    ],
    compiler_params=pltpu.CompilerParams(needs_layout_passes=False),
)
def _sc_scatter(g_lo, g_hi, src_hbm, dst_hbm, zrows_hbm,
                s_lo, s_hi, acc, sidx_s0, sidx_d0, sidx_s1, sidx_d1,
                idxd0, idxd1, rows0, rows1, gsem0, gsem1, isem):
    c = lax.axis_index("c")
    t = lax.axis_index("s")
    pltpu.sync_copy(zrows_hbm, acc.at[pl.ds(t * ROWS_PER_TILE, ROWS_PER_TILE)])
    ebase = t * EP_TILE
    sbufs = ((sidx_s0, sidx_d0), (sidx_s1, sidx_d1))
    pltpu.sync_copy(src_hbm.at[pl.ds(ebase, SC_E)], sidx_s0)
    pltpu.sync_copy(dst_hbm.at[pl.ds(ebase, SC_E)], sidx_d0)
    plsc.subcore_barrier()

    def run(g_ref):
        def fill_and_gather(sidx_s, sidx_d, chunk, idxd_b, rows_b, gsem_b):
            base = chunk * K_EDGE
            for j in range(K_EDGE // LANES):
                idxd_b[pl.ds(j * LANES, LANES)] = (
                    sidx_d[pl.ds(base + j * LANES, LANES)])
            pltpu.async_copy(
                g_ref.at[sidx_s.at[pl.ds(base, K_EDGE)]], rows_b, gsem_b)

        def wait_and_scatter(idxd_b, rows_b, gsem_b):
            pltpu.make_async_copy(
                g_ref.at[sidx_s0.at[pl.ds(0, K_EDGE)]], rows_b, gsem_b).wait()
            pltpu.sync_copy(rows_b, acc.at[idxd_b], add=True)

        for s in range(N_SUPER):
            sidx_s, sidx_d = sbufs[s % 2]
            nxt_s, nxt_d = sbufs[(s + 1) % 2]
            prefetch = s + 1 < N_SUPER
            if prefetch:
                off = pl.ds(ebase + (s + 1) * SC_E, SC_E)
                pf_s = pltpu.async_copy(src_hbm.at[off], nxt_s, isem)
                pf_d = pltpu.async_copy(dst_hbm.at[off], nxt_d, isem)

            fill_and_gather(sidx_s, sidx_d, 0, idxd0, rows0, gsem0)

            def body(j, _, sidx_s=sidx_s, sidx_d=sidx_d):
                fill_and_gather(sidx_s, sidx_d, 2 * j + 1, idxd1, rows1, gsem1)
                wait_and_scatter(idxd0, rows0, gsem0)

                @pl.when(j < CPS // 2 - 1)
                def _():
                    fill_and_gather(sidx_s, sidx_d, 2 * j + 2,
                                    idxd0, rows0, gsem0)

                wait_and_scatter(idxd1, rows1, gsem1)
                return 0

            lax.fori_loop(0, CPS // 2, body, 0)
            if prefetch:
                pf_s.wait()
                pf_d.wait()

    @pl.when(c == 0)
    def _():
        run(g_lo)

    @pl.when(c == 1)
    def _():
        run(g_hi)

    plsc.subcore_barrier()
    rb = pl.ds(t * ROWS_PER_TILE, ROWS_PER_TILE)

    @pl.when(c == 0)
    def _():
        pltpu.sync_copy(acc.at[rb], s_lo.at[rb])

    @pl.when(c == 1)
    def _():
        pltpu.sync_copy(acc.at[rb], s_hi.at[rb])


def _tc_pre_body(x_ref, cnt_ref, w_ref, glo_ref, ghi_ref, dinv_ref):
    deg = jnp.sum(cnt_ref[...], axis=1, keepdims=True) + 1.0
    dinv = lax.rsqrt(deg)
    h = jnp.dot(x_ref[...], w_ref[...], preferred_element_type=jnp.float32)
    g = dinv * h
    glo_ref[...] = g[:, :HALF]
    ghi_ref[...] = g[:, HALF:]
    dinv_ref[...] = dinv


def _tc_pre(x, counts_t, w0):
    return pl.pallas_call(
        _tc_pre_body,
        grid=(GRID_N,),
        in_specs=[
            pl.BlockSpec((BLK, D_IN), lambda i: (i, 0)),
            pl.BlockSpec((BLK, NW), lambda i: (i, 0)),
            pl.BlockSpec((D_IN, D_HID), lambda i: (0, 0)),
        ],
        out_specs=[
            pl.BlockSpec((BLK, HALF), lambda i: (i, 0)),
            pl.BlockSpec((BLK, HALF), lambda i: (i, 0)),
            pl.BlockSpec((BLK, 1), lambda i: (i, 0)),
        ],
        out_shape=[
            jax.ShapeDtypeStruct((N, HALF), jnp.float32),
            jax.ShapeDtypeStruct((N, HALF), jnp.float32),
            jax.ShapeDtypeStruct((N, 1), jnp.float32),
        ],
    )(x, counts_t, w0)


def _tc_mid_body(slo_ref, shi_ref, glo_ref, ghi_ref, dinv_ref, b_ref, w_ref,
                 olo_ref, ohi_ref):
    dinv = dinv_ref[...]
    lo = dinv * (slo_ref[...] + glo_ref[...]) + b_ref[0, :HALF]
    hi = dinv * (shi_ref[...] + ghi_ref[...]) + b_ref[0, HALF:]
    t = jnp.concatenate([lo, hi], axis=1)
    h = jnp.dot(t, w_ref[...], preferred_element_type=jnp.float32)
    g = dinv * h
    olo_ref[...] = g[:, :HALF]
    ohi_ref[...] = g[:, HALF:]


def _tc_mid(s_lo, s_hi, g_lo, g_hi, dinv, b_prev, w_next):
    return pl.pallas_call(
        _tc_mid_body,
        grid=(GRID_N,),
        in_specs=[
            pl.BlockSpec((BLK, HALF), lambda i: (i, 0)),
            pl.BlockSpec((BLK, HALF), lambda i: (i, 0)),
            pl.BlockSpec((BLK, HALF), lambda i: (i, 0)),
            pl.BlockSpec((BLK, HALF), lambda i: (i, 0)),
            pl.BlockSpec((BLK, 1), lambda i: (i, 0)),
            pl.BlockSpec((1, D_HID), lambda i: (0, 0)),
            pl.BlockSpec((D_HID, D_HID), lambda i: (0, 0)),
        ],
        out_specs=[
            pl.BlockSpec((BLK, HALF), lambda i: (i, 0)),
            pl.BlockSpec((BLK, HALF), lambda i: (i, 0)),
        ],
        out_shape=[
            jax.ShapeDtypeStruct((N, HALF), jnp.float32),
            jax.ShapeDtypeStruct((N, HALF), jnp.float32),
        ],
    )(s_lo, s_hi, g_lo, g_hi, dinv, b_prev, w_next)


def _tc_post_body(slo_ref, shi_ref, glo_ref, ghi_ref, dinv_ref, b_ref,
                  wlin_ref, blin_ref, bi_ref, h1_ref, psum, pcnt):
    i = pl.program_id(0)

    @pl.when(i == 0)
    def _():
        psum[...] = jnp.zeros_like(psum)
        pcnt[...] = jnp.zeros_like(pcnt)

    dinv = dinv_ref[...]
    lo = dinv * (slo_ref[...] + glo_ref[...]) + b_ref[0, :HALF]
    hi = dinv * (shi_ref[...] + ghi_ref[...]) + b_ref[0, HALF:]
    t = jnp.concatenate([lo, hi], axis=1)
    z = jnp.dot(t, wlin_ref[...], preferred_element_type=jnp.float32)
    z = jnp.maximum(z + blin_ref[0, :], 0.0)
    iota = lax.broadcasted_iota(jnp.int32, (BLK, B), 1)
    oh = (bi_ref[...] == iota).astype(jnp.float32)
    psum[...] += lax.dot_general(
        oh, z, (((0,), (0,)), ((), ())), preferred_element_type=jnp.float32
    )
    pcnt[...] += lax.dot_general(
        oh, jnp.ones((BLK, 1), jnp.float32), (((0,), (0,)), ((), ())),
        preferred_element_type=jnp.float32,
    )

    @pl.when(i == GRID_N - 1)
    def _():
        h1_ref[...] = psum[...] / jnp.maximum(pcnt[...], 1.0)


def _tc_post(s_lo, s_hi, g_lo, g_hi, dinv, b2, wlin, blin, bi):
    return pl.pallas_call(
        _tc_post_body,
        grid=(GRID_N,),
        in_specs=[
            pl.BlockSpec((BLK, HALF), lambda i: (i, 0)),
            pl.BlockSpec((BLK, HALF), lambda i: (i, 0)),
            pl.BlockSpec((BLK, HALF), lambda i: (i, 0)),
            pl.BlockSpec((BLK, HALF), lambda i: (i, 0)),
            pl.BlockSpec((BLK, 1), lambda i: (i, 0)),
            pl.BlockSpec((1, D_HID), lambda i: (0, 0)),
            pl.BlockSpec((D_HID, D_GOUT), lambda i: (0, 0)),
            pl.BlockSpec((1, D_GOUT), lambda i: (0, 0)),
            pl.BlockSpec((BLK, 1), lambda i: (i, 0)),
        ],
        out_specs=pl.BlockSpec((B, D_GOUT), lambda i: (0, 0)),
        out_shape=jax.ShapeDtypeStruct((B, D_GOUT), jnp.float32),
        scratch_shapes=[
            pltpu.VMEM((B, D_GOUT), jnp.float32),
            pltpu.VMEM((B, 1), jnp.float32),
        ],
    )(s_lo, s_hi, g_lo, g_hi, dinv, b2, wlin, blin, bi)


def _tc_head_body(h1_ref, mf_ref, m0w, m0b, m1w, m1b, m2w, m2b,
                  p0wg, p0wm, p0b, p1w, p1b, ow, ob, out_ref):
    h2 = jnp.maximum(
        jnp.dot(mf_ref[...], m0w[...], preferred_element_type=jnp.float32)
        + m0b[0, :], 0.0)
    h2 = jnp.maximum(
        jnp.dot(h2, m1w[...], preferred_element_type=jnp.float32)
        + m1b[0, :], 0.0)
    h2 = jnp.maximum(
        jnp.dot(h2, m2w[...], preferred_element_type=jnp.float32)
        + m2b[0, :], 0.0)
    h = (jnp.dot(h1_ref[...], p0wg[...], preferred_element_type=jnp.float32)
         + jnp.dot(h2, p0wm[...], preferred_element_type=jnp.float32)
         + p0b[0, :])
    h = jnp.maximum(h, 0.0)
    h = jnp.maximum(
        jnp.dot(h, p1w[...], preferred_element_type=jnp.float32)
        + p1b[0, :], 0.0)
    out_ref[...] = (
        jnp.dot(h, ow[...], preferred_element_type=jnp.float32) + ob[0, :])


def _tc_head(h1, mf, m0w, m0b, m1w, m1b, m2w, m2b, p0wg, p0wm, p0b,
             p1w, p1b, ow, ob):
    args = (h1, mf, m0w, m0b, m1w, m1b, m2w, m2b, p0wg, p0wm, p0b,
            p1w, p1b, ow, ob)
    return pl.pallas_call(
        _tc_head_body,
        in_specs=[pl.BlockSpec(a.shape, lambda: (0, 0)) for a in args],
        out_specs=pl.BlockSpec((B, 1), lambda: (0, 0)),
        out_shape=jax.ShapeDtypeStruct((B, 1), jnp.float32),
    )(*args)


def kernel(x, edge_index, batch_index, mol_features,
           gcn0_W, gcn0_b, gcn1_W, gcn1_b, gcn2_W, gcn2_b,
           gcnlin_W, gcnlin_b,
           mlp0_W, mlp0_b, mlp1_W, mlp1_b, mlp2_W, mlp2_b,
           pred0_W, pred0_b, pred1_W, pred1_b, out_W, out_b):
    src = edge_index[0]
    dst = edge_index[1]
    bi = batch_index.reshape(N, 1).astype(jnp.int32)
    zvec = jnp.zeros((N,), jnp.float32)
    zrows = jnp.zeros((ROWS_PER_TILE, HALF), jnp.float32)

    counts = _sc_degree(dst, zvec)
    g_lo, g_hi, dinv = _tc_pre(x, counts.reshape(NW, N).T, gcn0_W)

    for w_next, b_prev in ((gcn1_W, gcn0_b), (gcn2_W, gcn1_b)):
        s_lo, s_hi = _sc_scatter(g_lo, g_hi, src, dst, zrows)
        g_lo, g_hi = _tc_mid(s_lo, s_hi, g_lo, g_hi, dinv,
                             b_prev.reshape(1, -1), w_next)

    s_lo, s_hi = _sc_scatter(g_lo, g_hi, src, dst, zrows)
    h1 = _tc_post(s_lo, s_hi, g_lo, g_hi, dinv, gcn2_b.reshape(1, -1),
                  gcnlin_W, gcnlin_b.reshape(1, -1), bi)

    return _tc_head(
        h1, mol_features,
        mlp0_W, mlp0_b.reshape(1, -1), mlp1_W, mlp1_b.reshape(1, -1),
        mlp2_W, mlp2_b.reshape(1, -1),
        pred0_W[:D_GOUT], pred0_W[D_GOUT:], pred0_b.reshape(1, -1),
        pred1_W, pred1_b.reshape(1, -1), out_W, out_b.reshape(1, -1))


# speedup vs baseline: 22.2522x; 1.1341x over previous
import functools

import jax
import jax.numpy as jnp
from jax import lax
from jax.experimental import pallas as pl
from jax.experimental.pallas import tpu as pltpu
from jax.experimental.pallas import tpu_sc as plsc

N = 10000
E = 320000
B = 64
D_IN = 128
D_HID = 256
D_GOUT = 128
M_IN = 200
M_HID = 256
M_OUT = 64
P_HID = 256

NC = 2
NS = 16
NW = NC * NS
LANES = 16

BLK = 1000
GRID_N = N // BLK
HALF = D_HID // 2
K_EDGE = 80
SC_E = 4000
EP_TILE = E // NS
N_SUPER = EP_TILE // SC_E
CPS = SC_E // K_EDGE
EP_DEG = E // NW
NPAD = 10240
ROWS_PER_TILE = NPAD // NS

_mesh = plsc.VectorSubcoreMesh(
    core_axis_name="c", subcore_axis_name="s", num_cores=NC, num_subcores=NS
)


@functools.partial(
    pl.kernel,
    out_type=jax.ShapeDtypeStruct((NW * N,), jnp.float32),
    mesh=_mesh,
    scratch_types=[
        pltpu.VMEM((EP_DEG,), jnp.int32),
        pltpu.VMEM((N,), jnp.float32),
    ],
    compiler_params=pltpu.CompilerParams(needs_layout_passes=False),
)
def _sc_degree(dst_hbm, zvec_hbm, out_hbm, idx_v, cnt_v):
    wid = lax.axis_index("s") * NC + lax.axis_index("c")
    pltpu.sync_copy(zvec_hbm, cnt_v)
    pltpu.sync_copy(dst_hbm.at[pl.ds(wid * EP_DEG, EP_DEG)], idx_v)
    ones = jnp.ones((LANES,), jnp.float32)

    def body(i, _):
        idx = idx_v[pl.ds(i * LANES, LANES)]
        plsc.addupdate_scatter(cnt_v, [idx], ones)
        return 0

    lax.fori_loop(0, EP_DEG // LANES, body, 0)
    pltpu.sync_copy(cnt_v, out_hbm.at[pl.ds(wid * N, N)])


@functools.partial(
    pl.kernel,
    out_type=(
        jax.ShapeDtypeStruct((NPAD, HALF), jnp.float32),
        jax.ShapeDtypeStruct((NPAD, HALF), jnp.float32),
    ),
    mesh=_mesh,
    scratch_types=[
        pltpu.VMEM_SHARED((NPAD, HALF), jnp.float32),
        pltpu.VMEM((SC_E,), jnp.int32),
        pltpu.VMEM((SC_E,), jnp.int32),
        pltpu.VMEM((SC_E,), jnp.int32),
        pltpu.VMEM((SC_E,), jnp.int32),
        pltpu.VMEM((K_EDGE,), jnp.int32),
        pltpu.VMEM((K_EDGE,), jnp.int32),
        pltpu.VMEM((K_EDGE,), jnp.int32),
        pltpu.VMEM((K_EDGE, HALF), jnp.float32),
        pltpu.VMEM((K_EDGE, HALF), jnp.float32),
        pltpu.VMEM((K_EDGE, HALF), jnp.float32),
        pltpu.SemaphoreType.DMA,
        pltpu.SemaphoreType.DMA,
        pltpu.SemaphoreType.DMA,
        pltpu.SemaphoreType.DMA,
        pltpu.SemaphoreType.DMA,
        pltpu.SemaphoreType.DMA,
        pltpu.SemaphoreType.DMA,
    ],
    compiler_params=pltpu.CompilerParams(needs_layout_passes=False),
)
def _sc_scatter(g_lo, g_hi, src_hbm, dst_hbm, zrows_hbm,
                s_lo, s_hi, acc, sidx_s0, sidx_d0, sidx_s1, sidx_d1,
                idxd0, idxd1, idxd2, rows0, rows1, rows2,
                gsem0, gsem1, gsem2, ssem0, ssem1, ssem2, isem):
    c = lax.axis_index("c")
    t = lax.axis_index("s")
    pltpu.sync_copy(zrows_hbm, acc.at[pl.ds(t * ROWS_PER_TILE, ROWS_PER_TILE)])
    ebase = t * EP_TILE
    sbufs = ((sidx_s0, sidx_d0), (sidx_s1, sidx_d1))
    pltpu.sync_copy(src_hbm.at[pl.ds(ebase, SC_E)], sidx_s0)
    pltpu.sync_copy(dst_hbm.at[pl.ds(ebase, SC_E)], sidx_d0)
    plsc.subcore_barrier()

    idxd = (idxd0, idxd1, idxd2)
    rows = (rows0, rows1, rows2)
    gsem = (gsem0, gsem1, gsem2)
    ssem = (ssem0, ssem1, ssem2)

    def run(g_ref):
        def fill_and_gather(sidx_s, sidx_d, chunk, b):
            base = chunk * K_EDGE
            for j in range(K_EDGE // LANES):
                idxd[b][pl.ds(j * LANES, LANES)] = (
                    sidx_d[pl.ds(base + j * LANES, LANES)])
            pltpu.async_copy(
                g_ref.at[sidx_s.at[pl.ds(base, K_EDGE)]], rows[b], gsem[b])

        def wait_gather_scatter(b):
            pltpu.make_async_copy(
                g_ref.at[sidx_s0.at[pl.ds(0, K_EDGE)]], rows[b],
                gsem[b]).wait()
            pltpu.async_copy(rows[b], acc.at[idxd[b]], ssem[b], add=True)

        def drain_scatter(b):
            pltpu.make_async_copy(rows[b], acc.at[idxd[b]], ssem[b]).wait()

        for s in range(N_SUPER):
            sidx_s, sidx_d = sbufs[s % 2]
            prefetch = s + 1 < N_SUPER
            if prefetch:
                off = pl.ds(ebase + (s + 1) * SC_E, SC_E)
                nxt_s, nxt_d = sbufs[(s + 1) % 2]
                pf_s = pltpu.async_copy(src_hbm.at[off], nxt_s, isem)
                pf_d = pltpu.async_copy(dst_hbm.at[off], nxt_d, isem)

            fill_and_gather(sidx_s, sidx_d, 0, 0)
            fill_and_gather(sidx_s, sidx_d, 1, 1)
            wait_gather_scatter(0)
            fill_and_gather(sidx_s, sidx_d, 2, 2)
            wait_gather_scatter(1)

            def body(m, _, sidx_s=sidx_s, sidx_d=sidx_d):
                cb = 3 * m
                drain_scatter(0)
                fill_and_gather(sidx_s, sidx_d, cb + 3, 0)
                wait_gather_scatter(2)
                drain_scatter(1)
                fill_and_gather(sidx_s, sidx_d, cb + 4, 1)
                wait_gather_scatter(0)
                drain_scatter(2)
                fill_and_gather(sidx_s, sidx_d, cb + 5, 2)
                wait_gather_scatter(1)
                return 0

            lax.fori_loop(0, (CPS - 5) // 3, body, 0)
            wait_gather_scatter(2)
            drain_scatter(0)
            fill_and_gather(sidx_s, sidx_d, CPS - 2, 0)
            wait_gather_scatter(0)
            drain_scatter(1)
            fill_and_gather(sidx_s, sidx_d, CPS - 1, 1)
            wait_gather_scatter(1)
            drain_scatter(0)
            drain_scatter(1)
            drain_scatter(2)
            if prefetch:
                pf_s.wait()
                pf_d.wait()

    @pl.when(c == 0)
    def _():
        run(g_lo)

    @pl.when(c == 1)
    def _():
        run(g_hi)

    plsc.subcore_barrier()
    rb = pl.ds(t * ROWS_PER_TILE, ROWS_PER_TILE)

    @pl.when(c == 0)
    def _():
        pltpu.sync_copy(acc.at[rb], s_lo.at[rb])

    @pl.when(c == 1)
    def _():
        pltpu.sync_copy(acc.at[rb], s_hi.at[rb])


def _tc_pre_body(x_ref, cnt_ref, w_ref, glo_ref, ghi_ref, dinv_ref):
    deg = jnp.sum(cnt_ref[...], axis=1, keepdims=True) + 1.0
    dinv = lax.rsqrt(deg)
    h = jnp.dot(x_ref[...], w_ref[...], preferred_element_type=jnp.float32)
    g = dinv * h
    glo_ref[...] = g[:, :HALF]
    ghi_ref[...] = g[:, HALF:]
    dinv_ref[...] = dinv


def _tc_pre(x, counts_t, w0):
    return pl.pallas_call(
        _tc_pre_body,
        grid=(GRID_N,),
        in_specs=[
            pl.BlockSpec((BLK, D_IN), lambda i: (i, 0)),
            pl.BlockSpec((BLK, NW), lambda i: (i, 0)),
            pl.BlockSpec((D_IN, D_HID), lambda i: (0, 0)),
        ],
        out_specs=[
            pl.BlockSpec((BLK, HALF), lambda i: (i, 0)),
            pl.BlockSpec((BLK, HALF), lambda i: (i, 0)),
            pl.BlockSpec((BLK, 1), lambda i: (i, 0)),
        ],
        out_shape=[
            jax.ShapeDtypeStruct((N, HALF), jnp.float32),
            jax.ShapeDtypeStruct((N, HALF), jnp.float32),
            jax.ShapeDtypeStruct((N, 1), jnp.float32),
        ],
    )(x, counts_t, w0)


def _tc_mid_body(slo_ref, shi_ref, glo_ref, ghi_ref, dinv_ref, b_ref, w_ref,
                 olo_ref, ohi_ref):
    dinv = dinv_ref[...]
    lo = dinv * (slo_ref[...] + glo_ref[...]) + b_ref[0, :HALF]
    hi = dinv * (shi_ref[...] + ghi_ref[...]) + b_ref[0, HALF:]
    t = jnp.concatenate([lo, hi], axis=1)
    h = jnp.dot(t, w_ref[...], preferred_element_type=jnp.float32)
    g = dinv * h
    olo_ref[...] = g[:, :HALF]
    ohi_ref[...] = g[:, HALF:]


def _tc_mid(s_lo, s_hi, g_lo, g_hi, dinv, b_prev, w_next):
    return pl.pallas_call(
        _tc_mid_body,
        grid=(GRID_N,),
        in_specs=[
            pl.BlockSpec((BLK, HALF), lambda i: (i, 0)),
            pl.BlockSpec((BLK, HALF), lambda i: (i, 0)),
            pl.BlockSpec((BLK, HALF), lambda i: (i, 0)),
            pl.BlockSpec((BLK, HALF), lambda i: (i, 0)),
            pl.BlockSpec((BLK, 1), lambda i: (i, 0)),
            pl.BlockSpec((1, D_HID), lambda i: (0, 0)),
            pl.BlockSpec((D_HID, D_HID), lambda i: (0, 0)),
        ],
        out_specs=[
            pl.BlockSpec((BLK, HALF), lambda i: (i, 0)),
            pl.BlockSpec((BLK, HALF), lambda i: (i, 0)),
        ],
        out_shape=[
            jax.ShapeDtypeStruct((N, HALF), jnp.float32),
            jax.ShapeDtypeStruct((N, HALF), jnp.float32),
        ],
    )(s_lo, s_hi, g_lo, g_hi, dinv, b_prev, w_next)


def _tc_post_body(slo_ref, shi_ref, glo_ref, ghi_ref, dinv_ref, b_ref,
                  wlin_ref, blin_ref, bi_ref, h1_ref, psum, pcnt):
    i = pl.program_id(0)

    @pl.when(i == 0)
    def _():
        psum[...] = jnp.zeros_like(psum)
        pcnt[...] = jnp.zeros_like(pcnt)

    dinv = dinv_ref[...]
    lo = dinv * (slo_ref[...] + glo_ref[...]) + b_ref[0, :HALF]
    hi = dinv * (shi_ref[...] + ghi_ref[...]) + b_ref[0, HALF:]
    t = jnp.concatenate([lo, hi], axis=1)
    z = jnp.dot(t, wlin_ref[...], preferred_element_type=jnp.float32)
    z = jnp.maximum(z + blin_ref[0, :], 0.0)
    iota = lax.broadcasted_iota(jnp.int32, (BLK, B), 1)
    oh = (bi_ref[...] == iota).astype(jnp.float32)
    psum[...] += lax.dot_general(
        oh, z, (((0,), (0,)), ((), ())), preferred_element_type=jnp.float32
    )
    pcnt[...] += lax.dot_general(
        oh, jnp.ones((BLK, 1), jnp.float32), (((0,), (0,)), ((), ())),
        preferred_element_type=jnp.float32,
    )

    @pl.when(i == GRID_N - 1)
    def _():
        h1_ref[...] = psum[...] / jnp.maximum(pcnt[...], 1.0)


def _tc_post(s_lo, s_hi, g_lo, g_hi, dinv, b2, wlin, blin, bi):
    return pl.pallas_call(
        _tc_post_body,
        grid=(GRID_N,),
        in_specs=[
            pl.BlockSpec((BLK, HALF), lambda i: (i, 0)),
            pl.BlockSpec((BLK, HALF), lambda i: (i, 0)),
            pl.BlockSpec((BLK, HALF), lambda i: (i, 0)),
            pl.BlockSpec((BLK, HALF), lambda i: (i, 0)),
            pl.BlockSpec((BLK, 1), lambda i: (i, 0)),
            pl.BlockSpec((1, D_HID), lambda i: (0, 0)),
            pl.BlockSpec((D_HID, D_GOUT), lambda i: (0, 0)),
            pl.BlockSpec((1, D_GOUT), lambda i: (0, 0)),
            pl.BlockSpec((BLK, 1), lambda i: (i, 0)),
        ],
        out_specs=pl.BlockSpec((B, D_GOUT), lambda i: (0, 0)),
        out_shape=jax.ShapeDtypeStruct((B, D_GOUT), jnp.float32),
        scratch_shapes=[
            pltpu.VMEM((B, D_GOUT), jnp.float32),
            pltpu.VMEM((B, 1), jnp.float32),
        ],
    )(s_lo, s_hi, g_lo, g_hi, dinv, b2, wlin, blin, bi)


def _tc_head_body(h1_ref, mf_ref, m0w, m0b, m1w, m1b, m2w, m2b,
                  p0wg, p0wm, p0b, p1w, p1b, ow, ob, out_ref):
    h2 = jnp.maximum(
        jnp.dot(mf_ref[...], m0w[...], preferred_element_type=jnp.float32)
        + m0b[0, :], 0.0)
    h2 = jnp.maximum(
        jnp.dot(h2, m1w[...], preferred_element_type=jnp.float32)
        + m1b[0, :], 0.0)
    h2 = jnp.maximum(
        jnp.dot(h2, m2w[...], preferred_element_type=jnp.float32)
        + m2b[0, :], 0.0)
    h = (jnp.dot(h1_ref[...], p0wg[...], preferred_element_type=jnp.float32)
         + jnp.dot(h2, p0wm[...], preferred_element_type=jnp.float32)
         + p0b[0, :])
    h = jnp.maximum(h, 0.0)
    h = jnp.maximum(
        jnp.dot(h, p1w[...], preferred_element_type=jnp.float32)
        + p1b[0, :], 0.0)
    out_ref[...] = (
        jnp.dot(h, ow[...], preferred_element_type=jnp.float32) + ob[0, :])


def _tc_head(h1, mf, m0w, m0b, m1w, m1b, m2w, m2b, p0wg, p0wm, p0b,
             p1w, p1b, ow, ob):
    args = (h1, mf, m0w, m0b, m1w, m1b, m2w, m2b, p0wg, p0wm, p0b,
            p1w, p1b, ow, ob)
    return pl.pallas_call(
        _tc_head_body,
        in_specs=[pl.BlockSpec(a.shape, lambda: (0, 0)) for a in args],
        out_specs=pl.BlockSpec((B, 1), lambda: (0, 0)),
        out_shape=jax.ShapeDtypeStruct((B, 1), jnp.float32),
    )(*args)


def kernel(x, edge_index, batch_index, mol_features,
           gcn0_W, gcn0_b, gcn1_W, gcn1_b, gcn2_W, gcn2_b,
           gcnlin_W, gcnlin_b,
           mlp0_W, mlp0_b, mlp1_W, mlp1_b, mlp2_W, mlp2_b,
           pred0_W, pred0_b, pred1_W, pred1_b, out_W, out_b):
    src = edge_index[0]
    dst = edge_index[1]
    src_p, dst_p = src, dst
    bi = batch_index.reshape(N, 1).astype(jnp.int32)
    zvec = jnp.zeros((N,), jnp.float32)
    zrows = jnp.zeros((ROWS_PER_TILE, HALF), jnp.float32)

    counts = _sc_degree(dst, zvec)
    g_lo, g_hi, dinv = _tc_pre(x, counts.reshape(NW, N).T, gcn0_W)

    for w_next, b_prev in ((gcn1_W, gcn0_b), (gcn2_W, gcn1_b)):
        s_lo, s_hi = _sc_scatter(g_lo, g_hi, src_p, dst_p, zrows)
        g_lo, g_hi = _tc_mid(s_lo, s_hi, g_lo, g_hi, dinv,
                             b_prev.reshape(1, -1), w_next)

    s_lo, s_hi = _sc_scatter(g_lo, g_hi, src, dst, zrows)
    h1 = _tc_post(s_lo, s_hi, g_lo, g_hi, dinv, gcn2_b.reshape(1, -1),
                  gcnlin_W, gcnlin_b.reshape(1, -1), bi)

    return _tc_head(
        h1, mol_features,
        mlp0_W, mlp0_b.reshape(1, -1), mlp1_W, mlp1_b.reshape(1, -1),
        mlp2_W, mlp2_b.reshape(1, -1),
        pred0_W[:D_GOUT], pred0_W[D_GOUT:], pred0_b.reshape(1, -1),
        pred1_W, pred1_b.reshape(1, -1), out_W, out_b.reshape(1, -1))


# speedup vs baseline: 22.3030x; 1.0023x over previous
import functools

import jax
import jax.numpy as jnp
from jax import lax
from jax.experimental import pallas as pl
from jax.experimental.pallas import tpu as pltpu
from jax.experimental.pallas import tpu_sc as plsc

N = 10000
E = 320000
B = 64
D_IN = 128
D_HID = 256
D_GOUT = 128
M_IN = 200
M_HID = 256
M_OUT = 64
P_HID = 256

NC = 2
NS = 16
NW = NC * NS
LANES = 16

BLK = 1000
GRID_N = N // BLK
HALF = D_HID // 2
K_EDGE = 80
SC_E = 4000
EP_TILE = E // NS
N_SUPER = EP_TILE // SC_E
CPS = SC_E // K_EDGE
EP_DEG = E // NW
NPAD = 10240
ROWS_PER_TILE = NPAD // NS

_mesh = plsc.VectorSubcoreMesh(
    core_axis_name="c", subcore_axis_name="s", num_cores=NC, num_subcores=NS
)


@functools.partial(
    pl.kernel,
    out_type=jax.ShapeDtypeStruct((NW * N,), jnp.float32),
    mesh=_mesh,
    scratch_types=[
        pltpu.VMEM((EP_DEG,), jnp.int32),
        pltpu.VMEM((N,), jnp.float32),
    ],
    compiler_params=pltpu.CompilerParams(needs_layout_passes=False),
)
def _sc_degree(dst_hbm, zvec_hbm, out_hbm, idx_v, cnt_v):
    wid = lax.axis_index("s") * NC + lax.axis_index("c")
    pltpu.sync_copy(zvec_hbm, cnt_v)
    pltpu.sync_copy(dst_hbm.at[pl.ds(wid * EP_DEG, EP_DEG)], idx_v)
    ones = jnp.ones((LANES,), jnp.float32)

    def body(i, _):
        idx = idx_v[pl.ds(i * LANES, LANES)]
        plsc.addupdate_scatter(cnt_v, [idx], ones)
        return 0

    lax.fori_loop(0, EP_DEG // LANES, body, 0)
    pltpu.sync_copy(cnt_v, out_hbm.at[pl.ds(wid * N, N)])


@functools.partial(
    pl.kernel,
    out_type=(
        jax.ShapeDtypeStruct((NPAD, HALF), jnp.float32),
        jax.ShapeDtypeStruct((NPAD, HALF), jnp.float32),
    ),
    mesh=_mesh,
    scratch_types=[
        pltpu.VMEM_SHARED((NPAD, HALF), jnp.float32),
        pltpu.VMEM((SC_E,), jnp.int32),
        pltpu.VMEM((SC_E,), jnp.int32),
        pltpu.VMEM((SC_E,), jnp.int32),
        pltpu.VMEM((SC_E,), jnp.int32),
        pltpu.VMEM((K_EDGE,), jnp.int32),
        pltpu.VMEM((K_EDGE,), jnp.int32),
        pltpu.VMEM((K_EDGE,), jnp.int32),
        pltpu.VMEM((K_EDGE, HALF), jnp.float32),
        pltpu.VMEM((K_EDGE, HALF), jnp.float32),
        pltpu.VMEM((K_EDGE, HALF), jnp.float32),
        pltpu.SemaphoreType.DMA,
        pltpu.SemaphoreType.DMA,
        pltpu.SemaphoreType.DMA,
        pltpu.SemaphoreType.DMA,
        pltpu.SemaphoreType.DMA,
        pltpu.SemaphoreType.DMA,
        pltpu.SemaphoreType.DMA,
    ],
    compiler_params=pltpu.CompilerParams(needs_layout_passes=False),
)
def _sc_scatter(g_lo, g_hi, src_hbm, dst_hbm, zrows_hbm,
                s_lo, s_hi, acc, sidx_s0, sidx_d0, sidx_s1, sidx_d1,
                idxd0, idxd1, idxd2, rows0, rows1, rows2,
                gsem0, gsem1, gsem2, ssem0, ssem1, ssem2, isem):
    c = lax.axis_index("c")
    t = lax.axis_index("s")
    pltpu.sync_copy(zrows_hbm, acc.at[pl.ds(t * ROWS_PER_TILE, ROWS_PER_TILE)])
    ebase = t * EP_TILE
    sbufs = ((sidx_s0, sidx_d0), (sidx_s1, sidx_d1))
    pltpu.sync_copy(src_hbm.at[pl.ds(ebase, SC_E)], sidx_s0)
    pltpu.sync_copy(dst_hbm.at[pl.ds(ebase, SC_E)], sidx_d0)
    plsc.subcore_barrier()

    idxd = (idxd0, idxd1, idxd2)
    rows = (rows0, rows1, rows2)
    gsem = (gsem0, gsem1, gsem2)
    ssem = (ssem0, ssem1, ssem2)

    def run(g_ref):
        def fill_and_gather(sidx_s, sidx_d, chunk, b):
            base = chunk * K_EDGE
            for j in range(K_EDGE // LANES):
                idxd[b][pl.ds(j * LANES, LANES)] = (
                    sidx_d[pl.ds(base + j * LANES, LANES)])
            pltpu.async_copy(
                g_ref.at[sidx_s.at[pl.ds(base, K_EDGE)]], rows[b], gsem[b])

        def wait_gather_scatter(b):
            pltpu.make_async_copy(
                g_ref.at[sidx_s0.at[pl.ds(0, K_EDGE)]], rows[b],
                gsem[b]).wait()
            pltpu.async_copy(rows[b], acc.at[idxd[b]], ssem[b], add=True)

        def drain_scatter(b):
            pltpu.make_async_copy(rows[b], acc.at[idxd[b]], ssem[b]).wait()

        for s in range(N_SUPER):
            sidx_s, sidx_d = sbufs[s % 2]
            prefetch = s + 1 < N_SUPER
            if prefetch:
                off = pl.ds(ebase + (s + 1) * SC_E, SC_E)
                nxt_s, nxt_d = sbufs[(s + 1) % 2]
                pf_s = pltpu.async_copy(src_hbm.at[off], nxt_s, isem)
                pf_d = pltpu.async_copy(dst_hbm.at[off], nxt_d, isem)

            fill_and_gather(sidx_s, sidx_d, 0, 0)
            fill_and_gather(sidx_s, sidx_d, 1, 1)
            wait_gather_scatter(0)
            fill_and_gather(sidx_s, sidx_d, 2, 2)
            wait_gather_scatter(1)

            def body(m, _, sidx_s=sidx_s, sidx_d=sidx_d):
                cb = 3 * m
                drain_scatter(0)
                fill_and_gather(sidx_s, sidx_d, cb + 3, 0)
                wait_gather_scatter(2)
                drain_scatter(1)
                fill_and_gather(sidx_s, sidx_d, cb + 4, 1)
                wait_gather_scatter(0)
                drain_scatter(2)
                fill_and_gather(sidx_s, sidx_d, cb + 5, 2)
                wait_gather_scatter(1)
                return 0

            lax.fori_loop(0, (CPS - 5) // 3, body, 0)
            wait_gather_scatter(2)
            drain_scatter(0)
            fill_and_gather(sidx_s, sidx_d, CPS - 2, 0)
            wait_gather_scatter(0)
            drain_scatter(1)
            fill_and_gather(sidx_s, sidx_d, CPS - 1, 1)
            wait_gather_scatter(1)
            drain_scatter(0)
            drain_scatter(1)
            drain_scatter(2)
            if prefetch:
                pf_s.wait()
                pf_d.wait()

    @pl.when(c == 0)
    def _():
        run(g_lo)

    @pl.when(c == 1)
    def _():
        run(g_hi)

    plsc.subcore_barrier()
    rb = pl.ds(t * ROWS_PER_TILE, ROWS_PER_TILE)

    @pl.when(c == 0)
    def _():
        pltpu.sync_copy(acc.at[rb], s_lo.at[rb])

    @pl.when(c == 1)
    def _():
        pltpu.sync_copy(acc.at[rb], s_hi.at[rb])


def _tc_pre_body(x_ref, cnt_ref, w_ref, glo_ref, ghi_ref, dinv_ref):
    deg = jnp.sum(cnt_ref[...], axis=1, keepdims=True) + 1.0
    dinv = lax.rsqrt(deg)
    h = jnp.dot(x_ref[...], w_ref[...], preferred_element_type=jnp.float32)
    g = dinv * h
    glo_ref[...] = g[:, :HALF]
    ghi_ref[...] = g[:, HALF:]
    dinv_ref[...] = dinv


def _tc_pre(x, counts_t, w0):
    return pl.pallas_call(
        _tc_pre_body,
        grid=(GRID_N,),
        in_specs=[
            pl.BlockSpec((BLK, D_IN), lambda i: (i, 0)),
            pl.BlockSpec((BLK, NW), lambda i: (i, 0)),
            pl.BlockSpec((D_IN, D_HID), lambda i: (0, 0)),
        ],
        out_specs=[
            pl.BlockSpec((BLK, HALF), lambda i: (i, 0)),
            pl.BlockSpec((BLK, HALF), lambda i: (i, 0)),
            pl.BlockSpec((BLK, 1), lambda i: (i, 0)),
        ],
        out_shape=[
            jax.ShapeDtypeStruct((N, HALF), jnp.float32),
            jax.ShapeDtypeStruct((N, HALF), jnp.float32),
            jax.ShapeDtypeStruct((N, 1), jnp.float32),
        ],
    )(x, counts_t, w0)


def _tc_mid_body(slo_ref, shi_ref, glo_ref, ghi_ref, dinv_ref, b_ref, w_ref,
                 olo_ref, ohi_ref):
    dinv = dinv_ref[...]
    lo = dinv * (slo_ref[...] + glo_ref[...]) + b_ref[0, :HALF]
    hi = dinv * (shi_ref[...] + ghi_ref[...]) + b_ref[0, HALF:]
    t = jnp.concatenate([lo, hi], axis=1)
    h = jnp.dot(t, w_ref[...], preferred_element_type=jnp.float32)
    g = dinv * h
    olo_ref[...] = g[:, :HALF]
    ohi_ref[...] = g[:, HALF:]


def _tc_mid(s_lo, s_hi, g_lo, g_hi, dinv, b_prev, w_next):
    return pl.pallas_call(
        _tc_mid_body,
        grid=(GRID_N,),
        in_specs=[
            pl.BlockSpec((BLK, HALF), lambda i: (i, 0)),
            pl.BlockSpec((BLK, HALF), lambda i: (i, 0)),
            pl.BlockSpec((BLK, HALF), lambda i: (i, 0)),
            pl.BlockSpec((BLK, HALF), lambda i: (i, 0)),
            pl.BlockSpec((BLK, 1), lambda i: (i, 0)),
            pl.BlockSpec((1, D_HID), lambda i: (0, 0)),
            pl.BlockSpec((D_HID, D_HID), lambda i: (0, 0)),
        ],
        out_specs=[
            pl.BlockSpec((BLK, HALF), lambda i: (i, 0)),
            pl.BlockSpec((BLK, HALF), lambda i: (i, 0)),
        ],
        out_shape=[
            jax.ShapeDtypeStruct((N, HALF), jnp.float32),
            jax.ShapeDtypeStruct((N, HALF), jnp.float32),
        ],
    )(s_lo, s_hi, g_lo, g_hi, dinv, b_prev, w_next)


def _tc_post_body(slo_ref, shi_ref, glo_ref, ghi_ref, dinv_ref, b_ref,
                  wlin_ref, blin_ref, bi_ref, mf_ref, m0w, m0b, m1w, m1b,
                  m2w, m2b, p0wg, p0wm, p0b, p1w, p1b, ow, ob,
                  out_ref, psum, pcnt):
    i = pl.program_id(0)

    @pl.when(i == 0)
    def _():
        psum[...] = jnp.zeros_like(psum)
        pcnt[...] = jnp.zeros_like(pcnt)

    dinv = dinv_ref[...]
    lo = dinv * (slo_ref[...] + glo_ref[...]) + b_ref[0, :HALF]
    hi = dinv * (shi_ref[...] + ghi_ref[...]) + b_ref[0, HALF:]
    t = jnp.concatenate([lo, hi], axis=1)
    z = jnp.dot(t, wlin_ref[...], preferred_element_type=jnp.float32)
    z = jnp.maximum(z + blin_ref[0, :], 0.0)
    iota = lax.broadcasted_iota(jnp.int32, (BLK, B), 1)
    oh = (bi_ref[...] == iota).astype(jnp.float32)
    psum[...] += lax.dot_general(
        oh, z, (((0,), (0,)), ((), ())), preferred_element_type=jnp.float32
    )
    pcnt[...] += lax.dot_general(
        oh, jnp.ones((BLK, 1), jnp.float32), (((0,), (0,)), ((), ())),
        preferred_element_type=jnp.float32,
    )

    @pl.when(i == GRID_N - 1)
    def _():
        h1 = psum[...] / jnp.maximum(pcnt[...], 1.0)
        h2 = jnp.maximum(
            jnp.dot(mf_ref[...], m0w[...], preferred_element_type=jnp.float32)
            + m0b[0, :], 0.0)
        h2 = jnp.maximum(
            jnp.dot(h2, m1w[...], preferred_element_type=jnp.float32)
            + m1b[0, :], 0.0)
        h2 = jnp.maximum(
            jnp.dot(h2, m2w[...], preferred_element_type=jnp.float32)
            + m2b[0, :], 0.0)
        h = (jnp.dot(h1, p0wg[...], preferred_element_type=jnp.float32)
             + jnp.dot(h2, p0wm[...], preferred_element_type=jnp.float32)
             + p0b[0, :])
        h = jnp.maximum(h, 0.0)
        h = jnp.maximum(
            jnp.dot(h, p1w[...], preferred_element_type=jnp.float32)
            + p1b[0, :], 0.0)
        out_ref[...] = (
            jnp.dot(h, ow[...], preferred_element_type=jnp.float32)
            + ob[0, :])


def _tc_post(s_lo, s_hi, g_lo, g_hi, dinv, b2, wlin, blin, bi, *head_args):
    blocked = [
        pl.BlockSpec((BLK, HALF), lambda i: (i, 0)),
        pl.BlockSpec((BLK, HALF), lambda i: (i, 0)),
        pl.BlockSpec((BLK, HALF), lambda i: (i, 0)),
        pl.BlockSpec((BLK, HALF), lambda i: (i, 0)),
        pl.BlockSpec((BLK, 1), lambda i: (i, 0)),
        pl.BlockSpec((1, D_HID), lambda i: (0, 0)),
        pl.BlockSpec((D_HID, D_GOUT), lambda i: (0, 0)),
        pl.BlockSpec((1, D_GOUT), lambda i: (0, 0)),
        pl.BlockSpec((BLK, 1), lambda i: (i, 0)),
    ]
    blocked += [pl.BlockSpec(a.shape, lambda i: (0, 0)) for a in head_args]
    return pl.pallas_call(
        _tc_post_body,
        grid=(GRID_N,),
        in_specs=blocked,
        out_specs=pl.BlockSpec((B, 1), lambda i: (0, 0)),
        out_shape=jax.ShapeDtypeStruct((B, 1), jnp.float32),
        scratch_shapes=[
            pltpu.VMEM((B, D_GOUT), jnp.float32),
            pltpu.VMEM((B, 1), jnp.float32),
        ],
    )(s_lo, s_hi, g_lo, g_hi, dinv, b2, wlin, blin, bi, *head_args)


def kernel(x, edge_index, batch_index, mol_features,
           gcn0_W, gcn0_b, gcn1_W, gcn1_b, gcn2_W, gcn2_b,
           gcnlin_W, gcnlin_b,
           mlp0_W, mlp0_b, mlp1_W, mlp1_b, mlp2_W, mlp2_b,
           pred0_W, pred0_b, pred1_W, pred1_b, out_W, out_b):
    src = edge_index[0]
    dst = edge_index[1]
    src_p, dst_p = src, dst
    bi = batch_index.reshape(N, 1).astype(jnp.int32)
    zvec = jnp.zeros((N,), jnp.float32)
    zrows = jnp.zeros((ROWS_PER_TILE, HALF), jnp.float32)

    counts = _sc_degree(dst, zvec)
    g_lo, g_hi, dinv = _tc_pre(x, counts.reshape(NW, N).T, gcn0_W)

    for w_next, b_prev in ((gcn1_W, gcn0_b), (gcn2_W, gcn1_b)):
        s_lo, s_hi = _sc_scatter(g_lo, g_hi, src_p, dst_p, zrows)
        g_lo, g_hi = _tc_mid(s_lo, s_hi, g_lo, g_hi, dinv,
                             b_prev.reshape(1, -1), w_next)

    s_lo, s_hi = _sc_scatter(g_lo, g_hi, src, dst, zrows)
    return _tc_post(
        s_lo, s_hi, g_lo, g_hi, dinv, gcn2_b.reshape(1, -1),
        gcnlin_W, gcnlin_b.reshape(1, -1), bi,
        mol_features,
        mlp0_W, mlp0_b.reshape(1, -1), mlp1_W, mlp1_b.reshape(1, -1),
        mlp2_W, mlp2_b.reshape(1, -1),
        pred0_W[:D_GOUT], pred0_W[D_GOUT:], pred0_b.reshape(1, -1),
        pred1_W, pred1_b.reshape(1, -1), out_W, out_b.reshape(1, -1))
